# baseline (speedup 1.0000x reference)
import functools

import jax
import jax.numpy as jnp
from jax import lax
from jax.experimental import pallas as pl
from jax.experimental.pallas import tpu as pltpu
from jax.experimental.pallas import tpu_sc as plsc

VOCAB = 1000000
EMBED_DIM = 32
BATCH = 16384
SEQ = 50

NC = 2
NS = 16
NW = NC * NS
LANES = 16

ROWS_PER_W = BATCH // NW
IDX_PER_W = ROWS_PER_W * SEQ
NCH = ROWS_PER_W // LANES

_MESH = plsc.VectorSubcoreMesh(
    core_axis_name="c", subcore_axis_name="s", num_cores=NC, num_subcores=NS
)


@functools.partial(
    pl.kernel,
    out_type=jax.ShapeDtypeStruct((BATCH, EMBED_DIM), jnp.float32),
    mesh=_MESH,
    scratch_types=[
        pltpu.VMEM((ROWS_PER_W, SEQ), jnp.int32),
        pltpu.VMEM((SEQ, ROWS_PER_W), jnp.int32),
        pltpu.VMEM((ROWS_PER_W, EMBED_DIM), jnp.float32),
        pltpu.SemaphoreType.DMA,
        pltpu.SemaphoreType.DMA,
    ],
    compiler_params=pltpu.CompilerParams(
        use_tc_tiling_on_sc=False, needs_layout_passes=False),
)
def _embed_bag(idx_hbm, table_hbm, out_hbm, idx_v, idxt_v, acc_v, sem0, sem1):
    wid = lax.axis_index("s") * NC + lax.axis_index("c")
    base_b = wid * ROWS_PER_W
    scale = jnp.float32(1.0 / SEQ)

    pltpu.sync_copy(idx_hbm.at[pl.ds(base_b, ROWS_PER_W)], idx_v)

    lane_iota = lax.iota(jnp.int32, LANES)

    def tr_body(l, _):
        lvec = jnp.full((LANES,), 0, jnp.int32) + l
        for c in range(NCH):
            v = plsc.load_gather(idx_v, [lane_iota + (c * LANES), lvec])
            idxt_v[l, pl.ds(c * LANES, LANES)] = v
        return 0

    lax.fori_loop(0, SEQ, tr_body, 0)

    pltpu.async_copy(table_hbm.at[idxt_v.at[0]], acc_v, sem0).wait()
    for l in range(1, SEQ):
        pltpu.async_copy(table_hbm.at[idxt_v.at[l]], acc_v, sem1, add=True)
    for l in range(1, SEQ):
        pltpu.make_async_copy(table_hbm.at[idxt_v.at[l]], acc_v, sem1).wait()

    def scale_body(b, _):
        acc_v[b, pl.ds(0, LANES)] = acc_v[b, pl.ds(0, LANES)] * scale
        acc_v[b, pl.ds(LANES, LANES)] = acc_v[b, pl.ds(LANES, LANES)] * scale
        return 0

    lax.fori_loop(0, ROWS_PER_W, scale_body, 0)
    pltpu.sync_copy(acc_v, out_hbm.at[pl.ds(base_b, ROWS_PER_W)])


def kernel(inputs, table):
    return _embed_bag(jnp.flip(inputs, 1).astype(jnp.int32), table)


# speedup vs baseline: 1.0249x; 1.0249x over previous
import functools

import jax
import jax.numpy as jnp
from jax import lax
from jax.experimental import pallas as pl
from jax.experimental.pallas import tpu as pltpu
from jax.experimental.pallas import tpu_sc as plsc

VOCAB = 1000000
EMBED_DIM = 32
BATCH = 16384
SEQ = 50

NC = 2
NS = 16
NW = NC * NS
LANES = 16

ROWS_PER_W = BATCH // NW

_MESH = plsc.VectorSubcoreMesh(
    core_axis_name="c", subcore_axis_name="s", num_cores=NC, num_subcores=NS
)


@functools.partial(
    pl.kernel,
    out_type=jax.ShapeDtypeStruct((BATCH, EMBED_DIM), jnp.float32),
    mesh=_MESH,
    scratch_types=[
        pltpu.VMEM((SEQ, ROWS_PER_W), jnp.int32),
        pltpu.VMEM((ROWS_PER_W, EMBED_DIM), jnp.float32),
        pltpu.SemaphoreType.DMA,
        pltpu.SemaphoreType.DMA,
    ],
    compiler_params=pltpu.CompilerParams(use_tc_tiling_on_sc=False),
)
def _embed_bag(idx_hbm, table_hbm, out_hbm, idx_v, acc_v, sem0, sem1):
    wid = lax.axis_index("s") * NC + lax.axis_index("c")
    base_b = wid * ROWS_PER_W
    scale = jnp.float32(1.0 / SEQ)

    pltpu.sync_copy(idx_hbm.at[:, pl.ds(base_b, ROWS_PER_W)], idx_v)

    SB = 4
    SBR = ROWS_PER_W // SB

    for s in range(SB):
        pltpu.async_copy(
            table_hbm.at[idx_v.at[0, pl.ds(s * SBR, SBR)]],
            acc_v.at[pl.ds(s * SBR, SBR)], sem0)
    for s in range(SB):
        pltpu.make_async_copy(
            table_hbm.at[idx_v.at[0, pl.ds(s * SBR, SBR)]],
            acc_v.at[pl.ds(s * SBR, SBR)], sem0).wait()

    for l in range(1, SEQ):
        for s in range(SB):
            pltpu.async_copy(
                table_hbm.at[idx_v.at[l, pl.ds(s * SBR, SBR)]],
                acc_v.at[pl.ds(s * SBR, SBR)], sem1, add=True)
    for l in range(1, SEQ):
        for s in range(SB):
            pltpu.make_async_copy(
                table_hbm.at[idx_v.at[l, pl.ds(s * SBR, SBR)]],
                acc_v.at[pl.ds(s * SBR, SBR)], sem1).wait()

    def scale_body(b, _):
        acc_v[b, pl.ds(0, LANES)] = acc_v[b, pl.ds(0, LANES)] * scale
        acc_v[b, pl.ds(LANES, LANES)] = acc_v[b, pl.ds(LANES, LANES)] * scale
        return 0

    lax.fori_loop(0, ROWS_PER_W, scale_body, 0)
    pltpu.sync_copy(acc_v, out_hbm.at[pl.ds(base_b, ROWS_PER_W)])


def kernel(inputs, table):
    idx_t = inputs.astype(jnp.int32).T
    return _embed_bag(idx_t, table)
